# Initial kernel scaffold; baseline (speedup 1.0000x reference)
import jax
import jax.numpy as jnp
from jax.experimental import pallas as pl


def kernel(x, indices, W0, b0, W1, b1, W2, b2, W3, b3, Wf, bf):
    raise NotImplementedError("write your pallas kernel here")


# speedup vs baseline: 3.0876x; 3.0876x over previous
import functools
import jax
import jax.numpy as jnp
from jax import lax
from jax.experimental import pallas as pl
from jax.experimental.pallas import tpu as pltpu
from jax.experimental.pallas import tpu_sc as plsc

B, N, K = 8, 2048, 40
NW = 32
GCH = 128


def _leaky(z):
    return jnp.where(z >= 0, z, 0.2 * z)


@functools.cache
def _make_sc_gather(V, D, R):
    r_per_w = R // NW
    n_iter = r_per_w // GCH
    mesh = plsc.VectorSubcoreMesh(core_axis_name="c", subcore_axis_name="s")

    @functools.partial(
        pl.kernel,
        mesh=mesh,
        out_type=jax.ShapeDtypeStruct((R, D), jnp.float32),
        scratch_types=[
            pltpu.VMEM((GCH,), jnp.int32),
            pltpu.VMEM((GCH, D), jnp.float32),
            pltpu.SemaphoreType.DMA,
        ],
    )
    def gather_k(table_hbm, idx_hbm, out_hbm, idx_v, rows_v, sem):
        wid = lax.axis_index("s") * 2 + lax.axis_index("c")
        base = wid * r_per_w

        def body(i, carry):
            off = base + i * GCH
            pltpu.sync_copy(idx_hbm.at[pl.ds(off, GCH)], idx_v)
            pltpu.async_copy(table_hbm.at[idx_v], rows_v, sem).wait()
            pltpu.sync_copy(rows_v, out_hbm.at[pl.ds(off, GCH)])
            return carry

        lax.fori_loop(0, n_iter, body, 0)

    return gather_k


def _edgeconv_body(C, BR, g_ref, h_ref, w_ref, b_ref, o_ref):
    Cp = g_ref.shape[1]
    dout = w_ref.shape[0]
    w1 = w_ref[:, :C]
    w2 = w_ref[:, C:]
    if Cp != C:
        zpad = jnp.zeros((dout, Cp - C), jnp.float32)
        w1 = jnp.concatenate([w1, zpad], axis=1)
        w2 = jnp.concatenate([w2, zpad], axis=1)
    hc = h_ref[...]
    g3 = g_ref[...].reshape(BR, K, Cp)
    diff = (g3 - hc[:, None, :]).astype(jnp.bfloat16).reshape(BR * K, Cp)
    dn = (((1,), (1,)), ((), ()))
    z1 = jax.lax.dot_general(diff, w1.astype(jnp.bfloat16), dn,
                             preferred_element_type=jnp.float32)
    zc = jax.lax.dot_general(hc.astype(jnp.bfloat16), w2.astype(jnp.bfloat16), dn,
                             preferred_element_type=jnp.float32)
    z3 = z1.reshape(BR, K, dout) + (zc + b_ref[...])[:, None, :]
    o_ref[...] = jnp.max(_leaky(z3), axis=1)


def _edgeconv_tc(g, h, W, b, C):
    R, Cp = h.shape
    dout = W.shape[0]
    BR = 128
    return pl.pallas_call(
        functools.partial(_edgeconv_body, C, BR),
        grid=(R // BR,),
        in_specs=[
            pl.BlockSpec((BR * K, Cp), lambda i: (i, 0)),
            pl.BlockSpec((BR, Cp), lambda i: (i, 0)),
            pl.BlockSpec((dout, 2 * C), lambda i: (0, 0)),
            pl.BlockSpec((1, dout), lambda i: (0, 0)),
        ],
        out_specs=pl.BlockSpec((BR, dout), lambda i: (i, 0)),
        out_shape=jax.ShapeDtypeStruct((R, dout), jnp.float32),
    )(g, h, W, b[None, :])


def _edge_layer(table, gidx, W, b, C):
    V, Cp = table.shape
    g = _make_sc_gather(V, Cp, gidx.shape[0])(table, gidx)
    return _edgeconv_tc(g, table, W, b, C)


def _score_body(hb_ref, hall_ref, s_ref):
    hb = hb_ref[0]
    ha = hall_ref[0]
    dn = (((1,), (1,)), ((), ()))
    g = jax.lax.dot_general(hb.astype(jnp.bfloat16), ha.astype(jnp.bfloat16),
                            dn, preferred_element_type=jnp.float32)
    xxb = jnp.sum(hb * hb, axis=1)
    xxa = jnp.sum(ha * ha, axis=1)
    s_ref[0] = (2.0 * g - xxb[:, None]) - xxa[None, :]


def _scores(h3):
    C = h3.shape[2]
    BR = 256
    return pl.pallas_call(
        _score_body,
        grid=(B, N // BR),
        in_specs=[
            pl.BlockSpec((1, BR, C), lambda b, i: (b, i, 0)),
            pl.BlockSpec((1, N, C), lambda b, i: (b, 0, 0)),
        ],
        out_specs=pl.BlockSpec((1, BR, N), lambda b, i: (b, i, 0)),
        out_shape=jax.ShapeDtypeStruct((B, N, N), jnp.float32),
    )(h3, h3)


def _final_body(h1_ref, h2_ref, h3_ref, h4_ref, wf_ref, bf_ref, o_ref):
    i = pl.program_id(1)
    dn = (((1,), (1,)), ((), ()))
    wf = wf_ref[...].astype(jnp.bfloat16)
    z = jax.lax.dot_general(h1_ref[0].astype(jnp.bfloat16), wf[:, 0:64], dn,
                            preferred_element_type=jnp.float32)
    z += jax.lax.dot_general(h2_ref[0].astype(jnp.bfloat16), wf[:, 64:128], dn,
                             preferred_element_type=jnp.float32)
    z += jax.lax.dot_general(h3_ref[0].astype(jnp.bfloat16), wf[:, 128:256], dn,
                             preferred_element_type=jnp.float32)
    z += jax.lax.dot_general(h4_ref[0].astype(jnp.bfloat16), wf[:, 256:512], dn,
                             preferred_element_type=jnp.float32)
    z = _leaky(z + bf_ref[...])
    bm = jnp.max(z, axis=0)

    @pl.when(i == 0)
    def _():
        o_ref[...] = jnp.full_like(o_ref[...], -jnp.inf)

    o_ref[0] = jnp.maximum(o_ref[0], bm[None, :])


def _final(h1, h2, h3, h4, Wf, bf):
    BR = 512
    W_DIM = Wf.shape[0]
    return pl.pallas_call(
        _final_body,
        grid=(B, N // BR),
        in_specs=[
            pl.BlockSpec((1, BR, h.shape[2]), lambda b, i: (b, i, 0))
            for h in (h1, h2, h3, h4)
        ] + [
            pl.BlockSpec((W_DIM, 512), lambda b, i: (0, 0)),
            pl.BlockSpec((1, W_DIM), lambda b, i: (0, 0)),
        ],
        out_specs=pl.BlockSpec((1, 1, W_DIM), lambda b, i: (b, 0, 0)),
        out_shape=jax.ShapeDtypeStruct((B, 1, W_DIM), jnp.float32),
    )(h1, h2, h3, h4, Wf, bf[None, :]).reshape(B, W_DIM)


_GOFF = None


def _gidx(idx):
    off = (jnp.arange(B, dtype=jnp.int32) * N)[:, None, None]
    return (idx.astype(jnp.int32) + off).reshape(-1)


def _pad128(a):
    c = a.shape[-1]
    return a if c == 128 else jnp.pad(a, ((0, 0), (0, 128 - c)))


@jax.jit
def kernel(x, indices, W0, b0, W1, b1, W2, b2, W3, b3, Wf, bf):
    xpad = _pad128(x.reshape(B * N, 3))
    h1 = _edge_layer(xpad, _gidx(indices), W0, b0, 3)
    idx1 = jax.lax.top_k(_scores(h1.reshape(B, N, 64)), K)[1]
    h2 = _edge_layer(_pad128(h1), _gidx(idx1), W1, b1, 64)
    idx2 = jax.lax.top_k(_scores(h2.reshape(B, N, 64)), K)[1]
    h3 = _edge_layer(_pad128(h2), _gidx(idx2), W2, b2, 64)
    idx3 = jax.lax.top_k(_scores(h3.reshape(B, N, 128)), K)[1]
    h4 = _edge_layer(h3, _gidx(idx3), W3, b3, 128)
    return _final(h1.reshape(B, N, 64), h2.reshape(B, N, 64),
                  h3.reshape(B, N, 128), h4.reshape(B, N, 256), Wf, bf)


# speedup vs baseline: 6.9558x; 2.2528x over previous
import functools
import jax
import jax.numpy as jnp
from jax import lax
from jax.experimental import pallas as pl
from jax.experimental.pallas import tpu as pltpu
from jax.experimental.pallas import tpu_sc as plsc

B, N, K = 8, 2048, 40
NW = 32
GCH = 128


def _leaky(z):
    return jnp.where(z >= 0, z, 0.2 * z)


@functools.cache
def _make_sc_gather(V, D, R):
    r_per_w = R // NW
    n_iter = r_per_w // GCH
    mesh = plsc.VectorSubcoreMesh(core_axis_name="c", subcore_axis_name="s")

    @functools.partial(
        pl.kernel,
        mesh=mesh,
        out_type=jax.ShapeDtypeStruct((R, D), jnp.float32),
        scratch_types=[
            pltpu.VMEM((GCH,), jnp.int32),
            pltpu.VMEM((GCH, D), jnp.float32),
            pltpu.SemaphoreType.DMA,
        ],
    )
    def gather_k(table_hbm, idx_hbm, out_hbm, idx_v, rows_v, sem):
        wid = lax.axis_index("s") * 2 + lax.axis_index("c")
        base = wid * r_per_w

        def body(i, carry):
            off = base + i * GCH
            pltpu.sync_copy(idx_hbm.at[pl.ds(off, GCH)], idx_v)
            pltpu.async_copy(table_hbm.at[idx_v], rows_v, sem).wait()
            pltpu.sync_copy(rows_v, out_hbm.at[pl.ds(off, GCH)])
            return carry

        lax.fori_loop(0, n_iter, body, 0)

    return gather_k


def _edgeconv_body(C, BR, g_ref, h_ref, w_ref, b_ref, o_ref):
    Cp = g_ref.shape[1]
    dout = w_ref.shape[0]
    w1 = w_ref[:, :C]
    w2 = w_ref[:, C:]
    if Cp != C:
        zpad = jnp.zeros((dout, Cp - C), jnp.float32)
        w1 = jnp.concatenate([w1, zpad], axis=1)
        w2 = jnp.concatenate([w2, zpad], axis=1)
    hc = h_ref[...]
    g3 = g_ref[...].reshape(BR, K, Cp)
    diff = (g3 - hc[:, None, :]).astype(jnp.bfloat16).reshape(BR * K, Cp)
    dn = (((1,), (1,)), ((), ()))
    z1 = jax.lax.dot_general(diff, w1.astype(jnp.bfloat16), dn,
                             preferred_element_type=jnp.float32)
    zc = jax.lax.dot_general(hc.astype(jnp.bfloat16), w2.astype(jnp.bfloat16), dn,
                             preferred_element_type=jnp.float32)
    z3 = z1.reshape(BR, K, dout) + (zc + b_ref[...])[:, None, :]
    o_ref[...] = jnp.max(_leaky(z3), axis=1)


def _edgeconv_tc(g, h, W, b, C):
    R, Cp = h.shape
    dout = W.shape[0]
    BR = 128
    return pl.pallas_call(
        functools.partial(_edgeconv_body, C, BR),
        grid=(R // BR,),
        in_specs=[
            pl.BlockSpec((BR * K, Cp), lambda i: (i, 0)),
            pl.BlockSpec((BR, Cp), lambda i: (i, 0)),
            pl.BlockSpec((dout, 2 * C), lambda i: (0, 0)),
            pl.BlockSpec((1, dout), lambda i: (0, 0)),
        ],
        out_specs=pl.BlockSpec((BR, dout), lambda i: (i, 0)),
        out_shape=jax.ShapeDtypeStruct((R, dout), jnp.float32),
    )(g, h, W, b[None, :])


def _edge_layer(table, gidx, W, b, C):
    V, Cp = table.shape
    g = _make_sc_gather(V, Cp, gidx.shape[0])(table, gidx)
    return _edgeconv_tc(g, table, W, b, C)


def _score_tau_body(hb_ref, hall_ref, s_ref):
    BR = hb_ref.shape[1]
    hb = hb_ref[0]
    ha = hall_ref[0]
    dn = (((1,), (1,)), ((), ()))
    g = jax.lax.dot_general(hb.astype(jnp.bfloat16), ha.astype(jnp.bfloat16),
                            dn, preferred_element_type=jnp.float32)
    xxb = jnp.sum(hb * hb, axis=1)
    xxa = jnp.sum(ha * ha, axis=1)
    s = (2.0 * g - xxb[:, None]) - xxa[None, :]

    u = jax.lax.bitcast_convert_type(s, jnp.int32)
    key = u ^ ((u >> 31) & jnp.int32(0x7FFFFFFF))
    lo = jnp.min(key, axis=1, keepdims=True) - 1
    hi = jnp.max(key, axis=1, keepdims=True)

    def bs_body(_, lh):
        lo, hi = lh
        mid = (lo & hi) + ((lo ^ hi) >> 1)
        cnt = jnp.sum((key > mid).astype(jnp.int32), axis=1, keepdims=True)
        pred = cnt > K
        return jnp.where(pred, mid, lo), jnp.where(pred, hi, mid)

    lo, hi = lax.fori_loop(0, 32, bs_body, (lo, hi))
    tau_bits = hi ^ ((hi >> 31) & jnp.int32(0x7FFFFFFF))
    s_ref[0, :, :N] = s
    s_ref[0, :, N:] = jnp.broadcast_to(
        jax.lax.bitcast_convert_type(tau_bits, jnp.float32), (BR, 128))


def _scores_tau(h3):
    C = h3.shape[2]
    BR = 256
    return pl.pallas_call(
        _score_tau_body,
        grid=(B, N // BR),
        in_specs=[
            pl.BlockSpec((1, BR, C), lambda b, i: (b, i, 0)),
            pl.BlockSpec((1, N, C), lambda b, i: (b, 0, 0)),
        ],
        out_specs=pl.BlockSpec((1, BR, N + 128), lambda b, i: (b, i, 0)),
        out_shape=jax.ShapeDtypeStruct((B, N, N + 128), jnp.float32),
    )(h3, h3)


_SROW = N + 128


@functools.cache
def _make_sc_topk_gather():
    R = B * N
    r_per_w = R // NW
    pairs = r_per_w // 2
    mesh = plsc.VectorSubcoreMesh(core_axis_name="c", subcore_axis_name="s")

    @functools.partial(
        pl.kernel,
        mesh=mesh,
        out_type=jax.ShapeDtypeStruct((R * K, 128), jnp.float32),
        scratch_types=[
            pltpu.VMEM((2, _SROW), jnp.float32),
            pltpu.VMEM((128,), jnp.int32),
            pltpu.VMEM((2 * K, 128), jnp.float32),
            pltpu.VMEM((16,), jnp.int32),
            pltpu.SemaphoreType.DMA,
        ],
        compiler_params=pltpu.CompilerParams(needs_layout_passes=False),
    )
    def topk_gather_k(s_hbm, table_hbm, g_hbm, srow_v, idx_v, grow_v, cnt_v, sem):
        wid = lax.axis_index("s") * 2 + lax.axis_index("c")
        base = wid * r_per_w
        lanes = lax.iota(jnp.int32, 16)
        last = lanes * 0 + 15

        def _csum16(x):
            for k in (1, 2, 4, 8):
                src = jnp.maximum(lanes - k, 0)
                x = x + jnp.where(lanes >= k,
                                  x.at[src].get(mode="promise_in_bounds"), 0)
            return x

        def _splat_last(x):
            return x.at[last].get(mode="promise_in_bounds")

        def pair_body(p, carry):
            row0 = base + 2 * p
            gbase = (row0 // N) * N
            pltpu.sync_copy(s_hbm.at[pl.ds(row0, 2)], srow_v)

            def do_row(j, off0):
                tauv = srow_v[j, pl.ds(N, 16)]

                def chunk_gt(c, offv):
                    v = srow_v[j, pl.ds(c * 16, 16)]
                    m = v > tauv
                    vals = lanes + (c * 16 + gbase)
                    pc = _csum16(jnp.where(m, 1, 0))
                    plsc.store_scatter(idx_v, [pc + (offv - 1)], vals, mask=m)
                    return offv + _splat_last(pc)

                offv = lax.fori_loop(0, N // 16, chunk_gt, lanes * 0 + off0)
                cnt_v[...] = offv
                noff = cnt_v[...][0]

                def eq_fill(_):
                    def chunk_eq(c, offv):
                        v = srow_v[j, pl.ds(c * 16, 16)]
                        meq = v == tauv
                        rem = (off0 + K) - offv
                        pc = _csum16(jnp.where(meq, 1, 0))
                        keep = jnp.logical_and(meq, pc <= rem)
                        vals = lanes + (c * 16 + gbase)
                        plsc.store_scatter(idx_v, [pc + (offv - 1)], vals, mask=keep)
                        return offv + jnp.minimum(rem, _splat_last(pc))

                    lax.fori_loop(0, N // 16, chunk_eq, offv)
                    return 0

                lax.cond(noff < off0 + K, eq_fill, lambda _: 0, 0)
                return None

            do_row(0, 0)
            do_row(1, K)
            pltpu.async_copy(table_hbm.at[idx_v.at[pl.ds(0, 2 * K)]], grow_v,
                             sem).wait()
            pltpu.sync_copy(grow_v, g_hbm.at[pl.ds(row0 * K, 2 * K)])
            return carry

        lax.fori_loop(0, pairs, pair_body, 0)

    return topk_gather_k


def _knn_edge_layer(h_prev_pad, hC, W, b, C):
    sp = _scores_tau(hC).reshape(B * N, _SROW)
    g = _make_sc_topk_gather()(sp, h_prev_pad)
    return _edgeconv_tc(g, h_prev_pad, W, b, C)


def _final_body(h1_ref, h2_ref, h3_ref, h4_ref, wf_ref, bf_ref, o_ref):
    i = pl.program_id(1)
    dn = (((1,), (1,)), ((), ()))
    wf = wf_ref[...].astype(jnp.bfloat16)
    z = jax.lax.dot_general(h1_ref[0].astype(jnp.bfloat16), wf[:, 0:64], dn,
                            preferred_element_type=jnp.float32)
    z += jax.lax.dot_general(h2_ref[0].astype(jnp.bfloat16), wf[:, 64:128], dn,
                             preferred_element_type=jnp.float32)
    z += jax.lax.dot_general(h3_ref[0].astype(jnp.bfloat16), wf[:, 128:256], dn,
                             preferred_element_type=jnp.float32)
    z += jax.lax.dot_general(h4_ref[0].astype(jnp.bfloat16), wf[:, 256:512], dn,
                             preferred_element_type=jnp.float32)
    z = _leaky(z + bf_ref[...])
    bm = jnp.max(z, axis=0)

    @pl.when(i == 0)
    def _():
        o_ref[...] = jnp.full_like(o_ref[...], -jnp.inf)

    o_ref[0] = jnp.maximum(o_ref[0], bm[None, :])


def _final(h1, h2, h3, h4, Wf, bf):
    BR = 512
    W_DIM = Wf.shape[0]
    return pl.pallas_call(
        _final_body,
        grid=(B, N // BR),
        in_specs=[
            pl.BlockSpec((1, BR, h.shape[2]), lambda b, i: (b, i, 0))
            for h in (h1, h2, h3, h4)
        ] + [
            pl.BlockSpec((W_DIM, 512), lambda b, i: (0, 0)),
            pl.BlockSpec((1, W_DIM), lambda b, i: (0, 0)),
        ],
        out_specs=pl.BlockSpec((1, 1, W_DIM), lambda b, i: (b, 0, 0)),
        out_shape=jax.ShapeDtypeStruct((B, 1, W_DIM), jnp.float32),
    )(h1, h2, h3, h4, Wf, bf[None, :]).reshape(B, W_DIM)


_GOFF = None


def _gidx(idx):
    off = (jnp.arange(B, dtype=jnp.int32) * N)[:, None, None]
    return (idx.astype(jnp.int32) + off).reshape(-1)


def _pad128(a):
    c = a.shape[-1]
    return a if c == 128 else jnp.pad(a, ((0, 0), (0, 128 - c)))


@jax.jit
def kernel(x, indices, W0, b0, W1, b1, W2, b2, W3, b3, Wf, bf):
    xpad = _pad128(x.reshape(B * N, 3))
    h1 = _edge_layer(xpad, _gidx(indices), W0, b0, 3)
    h1p = _pad128(h1)
    h2 = _knn_edge_layer(h1p, h1.reshape(B, N, 64), W1, b1, 64)
    h2p = _pad128(h2)
    h3 = _knn_edge_layer(h2p, h2.reshape(B, N, 64), W2, b2, 64)
    h4 = _knn_edge_layer(h3, h3.reshape(B, N, 128), W3, b3, 128)
    return _final(h1.reshape(B, N, 64), h2.reshape(B, N, 64),
                  h3.reshape(B, N, 128), h4.reshape(B, N, 256), Wf, bf)


# speedup vs baseline: 9.9592x; 1.4318x over previous
import functools
import jax
import jax.numpy as jnp
from jax import lax
from jax.experimental import pallas as pl
from jax.experimental.pallas import tpu as pltpu
from jax.experimental.pallas import tpu_sc as plsc

B, N, K = 8, 2048, 40
NW = 32
GCH = 128


def _leaky(z):
    return jnp.where(z >= 0, z, 0.2 * z)


@functools.cache
def _make_sc_gather(V, D, R):
    r_per_w = R // NW
    n_iter = r_per_w // GCH
    mesh = plsc.VectorSubcoreMesh(core_axis_name="c", subcore_axis_name="s")

    @functools.partial(
        pl.kernel,
        mesh=mesh,
        out_type=jax.ShapeDtypeStruct((R, D), jnp.float32),
        scratch_types=[
            pltpu.VMEM((2, GCH), jnp.int32),
            pltpu.VMEM((2, GCH, D), jnp.float32),
            pltpu.SemaphoreType.DMA,
            pltpu.SemaphoreType.DMA,
            pltpu.SemaphoreType.DMA,
        ],
    )
    def gather_k(table_hbm, idx_hbm, out_hbm, idx_v, rows_v, semL, semG, semW):
        wid = lax.axis_index("s") * 2 + lax.axis_index("c")
        base = wid * r_per_w

        def fire_load(i):
            pltpu.async_copy(idx_hbm.at[pl.ds(base + i * GCH, GCH)],
                             idx_v.at[lax.rem(i, 2)], semL)

        def fire_gather(i):
            par = lax.rem(i, 2)
            pltpu.async_copy(table_hbm.at[idx_v.at[par]], rows_v.at[par], semG)

        def fire_write(i):
            pltpu.async_copy(rows_v.at[lax.rem(i, 2)],
                             out_hbm.at[pl.ds(base + i * GCH, GCH)], semW)

        def wait(src, dst, sem):
            pltpu.make_async_copy(src, dst, sem).wait()

        fire_load(0)

        def body(i, carry):
            wait(idx_hbm.at[pl.ds(0, GCH)], idx_v.at[0], semL)

            @pl.when(i >= 1)
            def _():
                wait(table_hbm.at[pl.ds(0, GCH)], rows_v.at[0], semG)
                fire_write(i - 1)

            @pl.when(i + 1 < n_iter)
            def _():
                fire_load(i + 1)

            @pl.when(i >= 2)
            def _():
                wait(rows_v.at[0], out_hbm.at[pl.ds(0, GCH)], semW)

            fire_gather(i)
            return carry

        lax.fori_loop(0, n_iter, body, 0)
        wait(table_hbm.at[pl.ds(0, GCH)], rows_v.at[0], semG)
        fire_write(n_iter - 1)
        wait(rows_v.at[0], out_hbm.at[pl.ds(0, GCH)], semW)
        wait(rows_v.at[0], out_hbm.at[pl.ds(0, GCH)], semW)

    return gather_k


def _edgeconv_body(C, BR, g_ref, h_ref, w_ref, b_ref, o_ref):
    Cp = g_ref.shape[1]
    dout = w_ref.shape[0]
    w1 = w_ref[:, :C]
    w2 = w_ref[:, C:]
    if Cp != C:
        zpad = jnp.zeros((dout, Cp - C), jnp.float32)
        w1 = jnp.concatenate([w1, zpad], axis=1)
        w2 = jnp.concatenate([w2, zpad], axis=1)
    hc = h_ref[...]
    g3 = g_ref[...].reshape(BR, K, Cp)
    diff = (g3 - hc[:, None, :]).astype(jnp.bfloat16).reshape(BR * K, Cp)
    dn = (((1,), (1,)), ((), ()))
    z1 = jax.lax.dot_general(diff, w1.astype(jnp.bfloat16), dn,
                             preferred_element_type=jnp.float32)
    zc = jax.lax.dot_general(hc.astype(jnp.bfloat16), w2.astype(jnp.bfloat16), dn,
                             preferred_element_type=jnp.float32)
    z3 = z1.reshape(BR, K, dout) + (zc + b_ref[...])[:, None, :]
    o_ref[...] = jnp.max(_leaky(z3), axis=1)


def _edgeconv_tc(g, h, W, b, C):
    R, Cp = h.shape
    dout = W.shape[0]
    BR = 128
    return pl.pallas_call(
        functools.partial(_edgeconv_body, C, BR),
        grid=(R // BR,),
        in_specs=[
            pl.BlockSpec((BR * K, Cp), lambda i: (i, 0)),
            pl.BlockSpec((BR, Cp), lambda i: (i, 0)),
            pl.BlockSpec((dout, 2 * C), lambda i: (0, 0)),
            pl.BlockSpec((1, dout), lambda i: (0, 0)),
        ],
        out_specs=pl.BlockSpec((BR, dout), lambda i: (i, 0)),
        out_shape=jax.ShapeDtypeStruct((R, dout), jnp.float32),
    )(g, h, W, b[None, :])


def _edge_layer(table, gidx, W, b, C):
    V, Cp = table.shape
    g = _make_sc_gather(V, Cp, gidx.shape[0])(table, gidx)
    return _edgeconv_tc(g, table, W, b, C)


def _score_pos_body(hb_ref, hall_ref, s_ref):
    BR = hb_ref.shape[1]
    hb = hb_ref[0]
    ha = hall_ref[0]
    dn = (((1,), (1,)), ((), ()))
    g = jax.lax.dot_general(hb.astype(jnp.bfloat16), ha.astype(jnp.bfloat16),
                            dn, preferred_element_type=jnp.float32)
    xxb = jnp.sum(hb * hb, axis=1)
    xxa = jnp.sum(ha * ha, axis=1)
    s = (2.0 * g - xxb[:, None]) - xxa[None, :]

    u = jax.lax.bitcast_convert_type(s, jnp.int32)
    key = u ^ ((u >> 31) & jnp.int32(0x7FFFFFFF))
    lo = jnp.min(key, axis=1, keepdims=True) - 1
    hi = jnp.max(key, axis=1, keepdims=True)

    def bs_body(_, lh):
        lo, hi = lh
        mid = (lo & hi) + ((lo ^ hi) >> 1)
        cnt = jnp.sum((key > mid).astype(jnp.int32), axis=1, keepdims=True)
        pred = cnt > K
        return jnp.where(pred, mid, lo), jnp.where(pred, hi, mid)

    lo, hi = lax.fori_loop(0, 32, bs_body, (lo, hi))
    m_gt = key > hi
    m_eq = key == hi

    ci = m_gt.astype(jnp.int32) + (m_eq.astype(jnp.int32) << 12)
    lane = jax.lax.broadcasted_iota(jnp.int32, (BR, N), 1) % 128
    intra = ci
    for k in (1, 2, 4, 8, 16, 32, 64):
        intra = intra + jnp.where(lane >= k, pltpu.roll(intra, k, 1), 0)

    nch = N // 128
    sel = (jax.lax.broadcasted_iota(jnp.int32, (N, nch), 0) // 128
           == jax.lax.broadcasted_iota(jnp.int32, (N, nch), 1))
    csum = jax.lax.dot_general(ci.astype(jnp.float32), sel.astype(jnp.float32),
                               (((1,), (0,)), ((), ())),
                               preferred_element_type=jnp.float32).astype(jnp.int32)
    incl = csum
    for t in (1, 2, 4, 8):
        incl = incl + jnp.concatenate(
            [jnp.zeros((BR, t), jnp.int32), incl[:, :nch - t]], axis=1)
    excl = incl - csum
    pos = intra + jnp.broadcast_to(excl[:, :, None],
                                   (BR, nch, 128)).reshape(BR, N)

    rank_gt = pos & jnp.int32(0xFFF)
    rank_eq = pos >> 12
    n1 = jnp.sum(m_gt.astype(jnp.int32), axis=1, keepdims=True)
    keep_eq = jnp.logical_and(m_eq, rank_eq <= (K - n1))
    slot = jnp.where(m_gt, rank_gt - 1, n1 + rank_eq - 1)
    s_ref[0] = jnp.where(jnp.logical_or(m_gt, keep_eq), slot + 1, 0)


def _scores_pos(h3):
    C = h3.shape[2]
    BR = 256
    return pl.pallas_call(
        _score_pos_body,
        grid=(B, N // BR),
        in_specs=[
            pl.BlockSpec((1, BR, C), lambda b, i: (b, i, 0)),
            pl.BlockSpec((1, N, C), lambda b, i: (b, 0, 0)),
        ],
        out_specs=pl.BlockSpec((1, BR, N), lambda b, i: (b, i, 0)),
        out_shape=jax.ShapeDtypeStruct((B, N, N), jnp.int32),
    )(h3, h3)


@functools.cache
def _make_sc_topk_gather():
    R = B * N
    r_per_w = R // NW
    pairs = r_per_w // 2
    mesh = plsc.VectorSubcoreMesh(core_axis_name="c", subcore_axis_name="s")

    @functools.partial(
        pl.kernel,
        mesh=mesh,
        out_type=jax.ShapeDtypeStruct((R * K, 128), jnp.float32),
        scratch_types=[
            pltpu.VMEM((2, 2, N), jnp.int32),
            pltpu.VMEM((2, 2 * K), jnp.int32),
            pltpu.VMEM((2, 2 * K, 128), jnp.float32),
            pltpu.SemaphoreType.DMA,
            pltpu.SemaphoreType.DMA,
            pltpu.SemaphoreType.DMA,
        ],
        compiler_params=pltpu.CompilerParams(needs_layout_passes=False),
    )
    def topk_gather_k(s_hbm, table_hbm, g_hbm, srow_v, idx_v, grow_v,
                      semL, semG, semW):
        wid = lax.axis_index("s") * 2 + lax.axis_index("c")
        base = wid * r_per_w
        lanes = lax.iota(jnp.int32, 16)

        def fire_load(p):
            par = lax.rem(p, 2)
            pltpu.async_copy(s_hbm.at[pl.ds(base + 2 * p, 2)],
                             srow_v.at[par], semL)

        def wait_load():
            pltpu.make_async_copy(s_hbm.at[pl.ds(0, 2)], srow_v.at[0],
                                  semL).wait()

        def fire_gather(p):
            par = lax.rem(p, 2)
            pltpu.async_copy(table_hbm.at[idx_v.at[par]], grow_v.at[par], semG)

        def wait_gather():
            pltpu.make_async_copy(table_hbm.at[pl.ds(0, 2 * K)], grow_v.at[0],
                                  semG).wait()

        def fire_write(p):
            par = lax.rem(p, 2)
            pltpu.async_copy(grow_v.at[par],
                             g_hbm.at[pl.ds((base + 2 * p) * K, 2 * K)], semW)

        def wait_write():
            pltpu.make_async_copy(grow_v.at[0], g_hbm.at[pl.ds(0, 2 * K)],
                                  semW).wait()

        fire_load(0)

        def pair_body(p, carry):
            par = lax.rem(p, 2)
            gbase = ((base + 2 * p) // N) * N
            wait_load()

            @pl.when(p + 1 < pairs)
            def _():
                fire_load(p + 1)

            def do_row(j, off0):
                def chunk(c, carry):
                    for cc in (2 * c, 2 * c + 1):
                        pv = srow_v[par, j, pl.ds(cc * 16, 16)]
                        m = pv > 0
                        vals = lanes + (cc * 16 + gbase)
                        plsc.store_scatter(idx_v.at[par], [pv + (off0 - 1)],
                                           vals, mask=m)
                    return carry

                lax.fori_loop(0, N // 32, chunk, 0)

            do_row(0, 0)
            do_row(1, K)

            @pl.when(p >= 2)
            def _():
                wait_write()

            fire_gather(p)

            @pl.when(p >= 1)
            def _():
                wait_gather()
                fire_write(p - 1)

            return carry

        lax.fori_loop(0, pairs, pair_body, 0)
        wait_gather()
        fire_write(pairs - 1)
        wait_write()
        wait_write()

    return topk_gather_k


def _knn_edge_layer(h_prev_pad, hC, W, b, C):
    sp = _scores_pos(hC).reshape(B * N, N)
    g = _make_sc_topk_gather()(sp, h_prev_pad)
    return _edgeconv_tc(g, h_prev_pad, W, b, C)


def _final_body(h1_ref, h2_ref, h3_ref, h4_ref, wf_ref, bf_ref, o_ref):
    i = pl.program_id(1)
    dn = (((1,), (1,)), ((), ()))
    wf = wf_ref[...].astype(jnp.bfloat16)
    z = jax.lax.dot_general(h1_ref[0].astype(jnp.bfloat16), wf[:, 0:64], dn,
                            preferred_element_type=jnp.float32)
    z += jax.lax.dot_general(h2_ref[0].astype(jnp.bfloat16), wf[:, 64:128], dn,
                             preferred_element_type=jnp.float32)
    z += jax.lax.dot_general(h3_ref[0].astype(jnp.bfloat16), wf[:, 128:256], dn,
                             preferred_element_type=jnp.float32)
    z += jax.lax.dot_general(h4_ref[0].astype(jnp.bfloat16), wf[:, 256:512], dn,
                             preferred_element_type=jnp.float32)
    z = _leaky(z + bf_ref[...])
    bm = jnp.max(z, axis=0)

    @pl.when(i == 0)
    def _():
        o_ref[...] = jnp.full_like(o_ref[...], -jnp.inf)

    o_ref[0] = jnp.maximum(o_ref[0], bm[None, :])


def _final(h1, h2, h3, h4, Wf, bf):
    BR = 512
    W_DIM = Wf.shape[0]
    return pl.pallas_call(
        _final_body,
        grid=(B, N // BR),
        in_specs=[
            pl.BlockSpec((1, BR, h.shape[2]), lambda b, i: (b, i, 0))
            for h in (h1, h2, h3, h4)
        ] + [
            pl.BlockSpec((W_DIM, 512), lambda b, i: (0, 0)),
            pl.BlockSpec((1, W_DIM), lambda b, i: (0, 0)),
        ],
        out_specs=pl.BlockSpec((1, 1, W_DIM), lambda b, i: (b, 0, 0)),
        out_shape=jax.ShapeDtypeStruct((B, 1, W_DIM), jnp.float32),
    )(h1, h2, h3, h4, Wf, bf[None, :]).reshape(B, W_DIM)


def _gidx(idx):
    off = (jnp.arange(B, dtype=jnp.int32) * N)[:, None, None]
    return (idx.astype(jnp.int32) + off).reshape(-1)


def _pad128(a):
    c = a.shape[-1]
    return a if c == 128 else jnp.pad(a, ((0, 0), (0, 128 - c)))


@jax.jit
def kernel(x, indices, W0, b0, W1, b1, W2, b2, W3, b3, Wf, bf):
    xpad = _pad128(x.reshape(B * N, 3))
    h1 = _edge_layer(xpad, _gidx(indices), W0, b0, 3)
    h1p = _pad128(h1)
    h2 = _knn_edge_layer(h1p, h1.reshape(B, N, 64), W1, b1, 64)
    h2p = _pad128(h2)
    h3 = _knn_edge_layer(h2p, h2.reshape(B, N, 64), W2, b2, 64)
    h4 = _knn_edge_layer(h3, h3.reshape(B, N, 128), W3, b3, 128)
    return _final(h1.reshape(B, N, 64), h2.reshape(B, N, 64),
                  h3.reshape(B, N, 128), h4.reshape(B, N, 256), Wf, bf)
